# baseline (speedup 1.0000x reference)
import jax
import jax.numpy as jnp
from jax import lax
from jax.experimental import pallas as pl
from jax.experimental.pallas import tpu as pltpu
from jax.experimental.pallas import tpu_sc as plsc

_NC = 2
_NS = 16
_NW = _NC * _NS
_CH = 16


def _make_sc_body(B, S, D):
    CHD = _CH * D
    s_per_w = S // _NW
    n_chunks = s_per_w // _CH
    n_pairs = n_chunks // 2
    T = n_chunks * B

    def body(x_hbm, emb_hbm, out_hbm, ebuf, xbuf, in_sem, out_sem, emb_sem):
        wid = lax.axis_index("s") * _NC + lax.axis_index("c")
        s_base = wid * s_per_w

        def x_off(c, b):
            return (b * S + s_base + c * _CH) * D

        def start_in(c, b, slot):
            pltpu.async_copy(
                x_hbm.at[pl.ds(x_off(c, b), CHD)], xbuf.at[slot],
                in_sem.at[slot])

        def wait_in(slot):
            pltpu.make_async_copy(
                x_hbm.at[pl.ds(0, CHD)], xbuf.at[slot],
                in_sem.at[slot]).wait()

        def start_out(c, b, slot):
            pltpu.async_copy(
                xbuf.at[slot], out_hbm.at[pl.ds(x_off(c, b), CHD)],
                out_sem.at[slot])

        def wait_out(slot):
            pltpu.make_async_copy(
                xbuf.at[slot], out_hbm.at[pl.ds(0, CHD)],
                out_sem.at[slot]).wait()

        def start_emb(c, slot):
            pltpu.async_copy(
                emb_hbm.at[pl.ds((s_base + c * _CH) * D, CHD)],
                ebuf.at[slot], emb_sem.at[slot])

        def wait_emb(slot):
            pltpu.make_async_copy(
                emb_hbm.at[pl.ds(0, CHD)], ebuf.at[slot],
                emb_sem.at[slot]).wait()

        start_emb(0, 0)
        start_emb(1, 1)
        start_in(0, 0, 0)
        start_in(0, 1, 1)

        def pair_body(gg, carry):
            for gi in range(2):
                c = gg * 2 + gi
                if True:
                    for k in range(B):
                        t_ge_2 = (gg > 0) | (gi > 0) | (k >= 2)
                        s2 = (k + 2) % 4
                        if k < 2:
                            c2, b2 = c, k + 2

                            @pl.when((gg > 0) | (gi > 0))
                            def _():
                                wait_out(s2)
                            start_in(c2, b2, s2)
                        else:
                            b2 = k - 2

                            @pl.when(c + 1 < n_chunks)
                            def _():
                                wait_out(s2)
                                start_in(c + 1, b2, s2)

                        wait_in(k)
                        if k == 0:
                            wait_emb(gi)

                        @plsc.parallel_loop(0, CHD, 16, unroll=8)
                        def _(i):
                            plsc.addupdate(
                                xbuf.at[k].at[pl.ds(i, 16)],
                                ebuf.at[gi][pl.ds(i, 16)])

                        start_out(c, k, k)

                        if k == B - 1:
                            @pl.when(c + 2 < n_chunks)
                            def _():
                                start_emb(c + 2, gi)
            return carry

        lax.fori_loop(0, n_pairs, pair_body, 0)

        for slot in range(4):
            wait_out(slot)

    return body


def kernel(x, emb):
    B, S, D = x.shape
    mesh = plsc.VectorSubcoreMesh(core_axis_name="c", subcore_axis_name="s")
    run = pl.kernel(
        _make_sc_body(B, S, D),
        mesh=mesh,
        out_type=jax.ShapeDtypeStruct((B * S * D,), x.dtype),
        scratch_types=[
            pltpu.VMEM((2, _CH * D), jnp.float32),
            pltpu.VMEM((4, _CH * D), jnp.float32),
            pltpu.SemaphoreType.DMA((4,)),
            pltpu.SemaphoreType.DMA((4,)),
            pltpu.SemaphoreType.DMA((2,)),
        ],
    )
    out = run(x.reshape(-1), emb.reshape(-1))
    return out.reshape(B, S, D)


# speedup vs baseline: 2.4952x; 2.4952x over previous
import jax
import jax.numpy as jnp
from jax import lax
from jax.experimental import pallas as pl
from jax.experimental.pallas import tpu as pltpu
from jax.experimental.pallas import tpu_sc as plsc

_NC = 2
_NS = 16
_NW = _NC * _NS
_CH = 16


def _make_sc_body(B, S, D):
    s_per_w = S // _NW
    n_chunks = s_per_w // _CH
    n_pairs = n_chunks // 2

    def body(x_hbm, emb_hbm, out_hbm, ebuf, xbuf, in_sem, out_sem, emb_sem):
        wid = lax.axis_index("s") * _NC + lax.axis_index("c")
        s_base = wid * s_per_w

        def start_in(c, b, slot):
            pltpu.async_copy(
                x_hbm.at[b, pl.ds(s_base + c * _CH, _CH), :],
                xbuf.at[slot], in_sem.at[slot])

        def wait_in(slot):
            pltpu.make_async_copy(
                x_hbm.at[0, pl.ds(0, _CH), :], xbuf.at[slot],
                in_sem.at[slot]).wait()

        def start_out(c, b, slot):
            pltpu.async_copy(
                xbuf.at[slot],
                out_hbm.at[b, pl.ds(s_base + c * _CH, _CH), :],
                out_sem.at[slot])

        def wait_out(slot):
            pltpu.make_async_copy(
                xbuf.at[slot], out_hbm.at[0, pl.ds(0, _CH), :],
                out_sem.at[slot]).wait()

        def start_emb(c, slot):
            pltpu.async_copy(
                emb_hbm.at[pl.ds(s_base + c * _CH, _CH), :],
                ebuf.at[slot], emb_sem.at[slot])

        def wait_emb(slot):
            pltpu.make_async_copy(
                emb_hbm.at[pl.ds(0, _CH), :], ebuf.at[slot],
                emb_sem.at[slot]).wait()

        start_emb(0, 0)
        start_emb(1, 1)
        start_in(0, 0, 0)
        start_in(0, 1, 1)

        def pair_body(gg, carry):
            for gi in range(2):
                c = gg * 2 + gi
                for k in range(B):
                    s2 = (k + 2) % 4
                    if k < 2:
                        @pl.when((gg > 0) | (gi > 0))
                        def _():
                            wait_out(s2)
                        start_in(c, k + 2, s2)
                    else:
                        @pl.when(c + 1 < n_chunks)
                        def _():
                            wait_out(s2)
                            start_in(c + 1, k - 2, s2)

                    wait_in(k)
                    if k == 0:
                        wait_emb(gi)

                    @plsc.parallel_loop(0, _CH, 1, unroll=2)
                    def _(i):
                        for j in range(D // 16):
                            sl = pl.ds(j * 16, 16)
                            plsc.addupdate(
                                xbuf.at[k, i, sl], ebuf.at[gi, i][sl])

                    start_out(c, k, k)

                    if k == B - 1:
                        @pl.when(c + 2 < n_chunks)
                        def _():
                            start_emb(c + 2, gi)
            return carry

        lax.fori_loop(0, n_pairs, pair_body, 0)

        for slot in range(4):
            wait_out(slot)

    return body


def kernel(x, emb):
    B, S, D = x.shape
    mesh = plsc.VectorSubcoreMesh(core_axis_name="c", subcore_axis_name="s")
    run = pl.kernel(
        _make_sc_body(B, S, D),
        mesh=mesh,
        out_type=jax.ShapeDtypeStruct((B, S, D), x.dtype),
        scratch_types=[
            pltpu.VMEM((2, _CH, D), jnp.float32),
            pltpu.VMEM((4, _CH, D), jnp.float32),
            pltpu.SemaphoreType.DMA((4,)),
            pltpu.SemaphoreType.DMA((4,)),
            pltpu.SemaphoreType.DMA((2,)),
        ],
    )
    return run(x, emb)


# speedup vs baseline: 4.8392x; 1.9394x over previous
import jax
import jax.numpy as jnp
from jax.experimental import pallas as pl


_SEQ_TILE = 2048


def _add_kernel(x_ref, e_ref, o_ref):
    o_ref[...] = x_ref[...] + e_ref[...]


def kernel(x, emb):
    B, S, D = x.shape
    ts = _SEQ_TILE
    grid = (S // ts, B)
    return pl.pallas_call(
        _add_kernel,
        grid=grid,
        in_specs=[
            pl.BlockSpec((1, ts, D), lambda s, b: (b, s, 0)),
            pl.BlockSpec((ts, D), lambda s, b: (s, 0)),
        ],
        out_specs=pl.BlockSpec((1, ts, D), lambda s, b: (b, s, 0)),
        out_shape=jax.ShapeDtypeStruct(x.shape, x.dtype),
    )(x, emb)
